# baseline (speedup 1.0000x reference)
import jax
import jax.numpy as jnp
from jax.experimental import pallas as pl
from jax.experimental.pallas import tpu as pltpu

_N_CHUNKS = 8


def _dma_body(x_ref, o_ref, *sems):
    rows = x_ref.shape[0]
    chunk = rows // _N_CHUNKS
    copies = []
    for i in range(_N_CHUNKS):
        lo = i * chunk
        hi = rows if i == _N_CHUNKS - 1 else lo + chunk
        cp = pltpu.make_async_copy(
            x_ref.at[pl.ds(lo, hi - lo), :],
            o_ref.at[pl.ds(lo, hi - lo), :],
            sems[i],
        )
        cp.start()
        copies.append(cp)
    for cp in copies:
        cp.wait()


def kernel(logits):
    B, T, V = logits.shape
    x = logits.reshape(B * T, V)
    out = pl.pallas_call(
        _dma_body,
        in_specs=[pl.BlockSpec(memory_space=pl.ANY)],
        out_specs=pl.BlockSpec(memory_space=pl.ANY),
        out_shape=jax.ShapeDtypeStruct((B * T, V), logits.dtype),
        scratch_shapes=[pltpu.SemaphoreType.DMA] * _N_CHUNKS,
    )(x)
    return out.reshape(B, T, V)


# speedup vs baseline: 49.0521x; 49.0521x over previous
import jax
import jax.numpy as jnp
from jax.experimental import pallas as pl


def _copy_body(x_ref, o_ref):
    o_ref[...] = x_ref[...]


def kernel(logits):
    B, T, V = logits.shape
    rows = B * T
    x = logits.reshape(rows, V)
    bt = min(64, rows)
    out = pl.pallas_call(
        _copy_body,
        grid=(pl.cdiv(rows, bt),),
        in_specs=[pl.BlockSpec((bt, V), lambda i: (i, 0))],
        out_specs=pl.BlockSpec((bt, V), lambda i: (i, 0)),
        out_shape=jax.ShapeDtypeStruct((rows, V), logits.dtype),
    )(x)
    return out.reshape(B, T, V)
